# baseline (speedup 1.0000x reference)
import jax
import jax.numpy as jnp
import numpy as np
from jax import lax
from jax.experimental import pallas as pl
from jax.experimental.pallas import tpu as pltpu
from jax.experimental.pallas import tpu_sc as plsc

N = 10000
E = 320000
D = 128
AVG_NEIGH = 32.0

NC = 2
NS = 16
EW = E // (NC * NS)
K = 80
NCHUNK = EW // K
ZROWS = K
NZCHUNK = N // ZROWS
FROWS = 200
NFCHUNK = N // FROWS


_COLMAP = np.empty((D,), np.int32)
for _q in range(D // 32):
    for _i in range(16):
        _COLMAP[32 * _q + 2 * _i] = 32 * _q + _i
        _COLMAP[32 * _q + 2 * _i + 1] = 32 * _q + 16 + _i


def _h_body(x_ref, w_ref, o_ref):
    o_ref[...] = jnp.dot(x_ref[...], w_ref[...], preferred_element_type=jnp.float32)


def _radial_body(emb_ref, attr_ref, wr1_ref, wr2_ref, o_ref):
    t = jnp.dot(emb_ref[...], wr1_ref[...], preferred_element_type=jnp.float32)
    t = t * jax.nn.sigmoid(t)
    r = jnp.dot(t, wr2_ref[...], preferred_element_type=jnp.float32)
    o_ref[...] = pltpu.bitcast((r * attr_ref[...]).astype(jnp.bfloat16), jnp.int32)


def _post_body(x_ref, a0_ref, a1_ref, w2_ref, wsc_ref, o_ref):
    agg = (a0_ref[...] + a1_ref[...]) * (1.0 / AVG_NEIGH)
    conv = (jnp.dot(x_ref[...], wsc_ref[...], preferred_element_type=jnp.float32)
            + jnp.dot(agg, w2_ref[...], preferred_element_type=jnp.float32))
    o_ref[...] = x_ref[...] + conv * jax.nn.sigmoid(conv)


def _edge_body(h_hbm, src_hbm, dst_hbm, rad_hbm, out_hbm,
               srcb0, srcb1, dstb0, dstb1, rows0, rows1, radb0, radb1,
               acc,
               gsem0, gsem1, rsem0, rsem1, isem0, isem1):
    c = lax.axis_index("c")
    s = lax.axis_index("s")
    tbase = (c * NS + s) * EW
    rbase = (c * NS + s) * (EW // 2)

    srcb = (srcb0, srcb1)
    dstb = (dstb0, dstb1)
    rows = (rows0, rows1)
    radb = (radb0, radb1)
    gsem = (gsem0, gsem1)
    rsem = (rsem0, rsem1)
    isem = (isem0, isem1)

    def _zrow(j, _):
        for q in range(D // 16):
            rows0[j, pl.ds(q * 16, 16)] = jnp.zeros((16,), jnp.float32)
        return 0
    lax.fori_loop(0, ZROWS, _zrow, 0)
    for t in range((NZCHUNK + NS - 1) // NS):
        cid = t * NS + s
        @pl.when(cid < NZCHUNK)
        def _():
            pltpu.sync_copy(rows0, acc.at[pl.ds(cid * ZROWS, ZROWS)])
    plsc.subcore_barrier()

    def _issue_gr(i, sl):
        pltpu.async_copy(h_hbm.at[srcb[sl]], rows[sl], gsem[sl])
        pltpu.async_copy(rad_hbm.at[pl.ds(rbase + i * (K // 2), K // 2)], radb[sl], rsem[sl])

    def _issue_idx(i, sl):
        eb = tbase + i * K
        pltpu.async_copy(src_hbm.at[pl.ds(eb, K)], srcb[sl], isem[sl])
        pltpu.async_copy(dst_hbm.at[pl.ds(eb, K)], dstb[sl], isem[sl])

    pltpu.sync_copy(src_hbm.at[pl.ds(tbase, K)], srcb0)
    pltpu.sync_copy(dst_hbm.at[pl.ds(tbase, K)], dstb0)
    _issue_idx(1, 1)
    _issue_gr(0, 0)

    def _step(i, sl):
        nsl = 1 - sl
        pltpu.make_async_copy(h_hbm.at[srcb[sl]], rows[sl], gsem[sl]).wait()
        pltpu.make_async_copy(rad_hbm.at[pl.ds(rbase, K // 2)], radb[sl], rsem[sl]).wait()

        @pl.when(i + 1 < NCHUNK)
        def _():
            pltpu.make_async_copy(src_hbm.at[pl.ds(tbase, K)], srcb[nsl], isem[nsl]).wait()
            pltpu.make_async_copy(dst_hbm.at[pl.ds(tbase, K)], dstb[nsl], isem[nsl]).wait()
            _issue_gr(i + 1, nsl)

        @plsc.parallel_loop(0, K // 2, unroll=8)
        def _mul(jp):
            for q in range(D // 16):
                qs = pl.ds(q * 16, 16)
                rv = radb[sl][jp, qs]
                ra = jax.lax.bitcast_convert_type(jnp.left_shift(rv, 16), jnp.float32)
                rb = jax.lax.bitcast_convert_type(jnp.bitwise_and(rv, jnp.int32(-65536)), jnp.float32)
                rows[sl][2 * jp, qs] = rows[sl][2 * jp, qs] * ra
                rows[sl][2 * jp + 1, qs] = rows[sl][2 * jp + 1, qs] * rb
        pltpu.sync_copy(rows[sl], acc.at[dstb[sl]], add=True)

        @pl.when(i + 2 < NCHUNK)
        def _():
            _issue_idx(i + 2, sl)

    def _chunk(i, _):
        @pl.when(i % 2 == 0)
        def _():
            _step(i, 0)

        @pl.when(i % 2 == 1)
        def _():
            _step(i, 1)
        return 0
    lax.fori_loop(0, NCHUNK, _chunk, 0)

    plsc.subcore_barrier()
    for t in range((NFCHUNK + NS - 1) // NS):
        cid = t * NS + s
        @pl.when(cid < NFCHUNK)
        def _():
            r0 = cid * FROWS
            pltpu.sync_copy(acc.at[pl.ds(r0, FROWS)], out_hbm.at[pl.ds(c * N + r0, FROWS)])


def kernel(node_features, edge_index, edge_attrs, edge_embedding, W1, W2, W_sc, Wr1, Wr2):
    x = node_features
    src = edge_index[0]
    dst = edge_index[1]

    h = pl.pallas_call(
        _h_body,
        grid=(5,),
        in_specs=[pl.BlockSpec((2000, D), lambda i: (i, 0)),
                  pl.BlockSpec((D, D), lambda i: (0, 0))],
        out_specs=pl.BlockSpec((2000, D), lambda i: (i, 0)),
        out_shape=jax.ShapeDtypeStruct((N, D), jnp.float32),
    )(x, W1)

    BE = 8000
    radial = pl.pallas_call(
        _radial_body,
        grid=(E // BE,),
        in_specs=[pl.BlockSpec((BE, 16), lambda i: (i, 0)),
                  pl.BlockSpec((BE, 1), lambda i: (i, 0)),
                  pl.BlockSpec((16, 64), lambda i: (0, 0)),
                  pl.BlockSpec((64, D), lambda i: (0, 0))],
        out_specs=pl.BlockSpec((BE // 2, D), lambda i: (i, 0)),
        out_shape=jax.ShapeDtypeStruct((E // 2, D), jnp.int32),
    )(edge_embedding, edge_attrs, Wr1, Wr2)

    agg2 = pl.kernel(
        _edge_body,
        out_type=jax.ShapeDtypeStruct((2 * N, D), jnp.float32),
        mesh=plsc.VectorSubcoreMesh(core_axis_name="c", subcore_axis_name="s"),
        scratch_types=[
            pltpu.VMEM((K,), jnp.int32),
            pltpu.VMEM((K,), jnp.int32),
            pltpu.VMEM((K,), jnp.int32),
            pltpu.VMEM((K,), jnp.int32),
            pltpu.VMEM((K, D), jnp.float32),
            pltpu.VMEM((K, D), jnp.float32),
            pltpu.VMEM((K // 2, D), jnp.int32),
            pltpu.VMEM((K // 2, D), jnp.int32),
            pltpu.VMEM_SHARED((N, D), jnp.float32),
            pltpu.SemaphoreType.DMA,
            pltpu.SemaphoreType.DMA,
            pltpu.SemaphoreType.DMA,
            pltpu.SemaphoreType.DMA,
            pltpu.SemaphoreType.DMA,
            pltpu.SemaphoreType.DMA,
        ],
    )(h, src, dst, radial)

    out = pl.pallas_call(
        _post_body,
        grid=(5,),
        in_specs=[pl.BlockSpec((2000, D), lambda i: (i, 0)),
                  pl.BlockSpec((2000, D), lambda i: (i, 0)),
                  pl.BlockSpec((2000, D), lambda i: (i, 0)),
                  pl.BlockSpec((D, D), lambda i: (0, 0)),
                  pl.BlockSpec((D, D), lambda i: (0, 0))],
        out_specs=pl.BlockSpec((2000, D), lambda i: (i, 0)),
        out_shape=jax.ShapeDtypeStruct((N, D), jnp.float32),
    )(x, agg2[:N], agg2[N:], W2, W_sc)
    return out


# speedup vs baseline: 1.0103x; 1.0103x over previous
import jax
import jax.numpy as jnp
import numpy as np
from jax import lax
from jax.experimental import pallas as pl
from jax.experimental.pallas import tpu as pltpu
from jax.experimental.pallas import tpu_sc as plsc

N = 10000
E = 320000
D = 128
AVG_NEIGH = 32.0

NC = 2
NS = 16
EW = E // (NC * NS)
K = 80
NCHUNK = EW // K
ZROWS = K
NZCHUNK = N // ZROWS
FROWS = 200
NFCHUNK = N // FROWS


_COLMAP = np.empty((D,), np.int32)
for _q in range(D // 32):
    for _i in range(16):
        _COLMAP[32 * _q + 2 * _i] = 32 * _q + _i
        _COLMAP[32 * _q + 2 * _i + 1] = 32 * _q + 16 + _i


def _h_body(x_ref, w_ref, o_ref):
    o_ref[...] = jnp.dot(x_ref[...], w_ref[...], preferred_element_type=jnp.float32)


def _radial_body(emb_ref, attr_ref, wr1_ref, wr2_ref, o_ref):
    t = jnp.dot(emb_ref[...], wr1_ref[...], preferred_element_type=jnp.float32)
    t = t * jax.nn.sigmoid(t)
    r = jnp.dot(t, wr2_ref[...], preferred_element_type=jnp.float32)
    o_ref[...] = pltpu.bitcast((r * attr_ref[...]).astype(jnp.bfloat16), jnp.int32)


def _post_body(x_ref, a0_ref, a1_ref, w2_ref, wsc_ref, o_ref):
    agg = (a0_ref[...] + a1_ref[...]) * (1.0 / AVG_NEIGH)
    conv = (jnp.dot(x_ref[...], wsc_ref[...], preferred_element_type=jnp.float32)
            + jnp.dot(agg, w2_ref[...], preferred_element_type=jnp.float32))
    o_ref[...] = x_ref[...] + conv * jax.nn.sigmoid(conv)


def _edge_body(h_hbm, src_hbm, dst_hbm, rad_hbm, out_hbm,
               srcb0, srcb1, dstb0, dstb1, rows0, rows1, radb0, radb1,
               acc,
               gsem0, gsem1, rsem0, rsem1, isem0, isem1):
    c = lax.axis_index("c")
    s = lax.axis_index("s")
    tbase = (c * NS + s) * EW
    rbase = (c * NS + s) * (EW // 2)

    srcb = (srcb0, srcb1)
    dstb = (dstb0, dstb1)
    rows = (rows0, rows1)
    radb = (radb0, radb1)
    gsem = (gsem0, gsem1)
    rsem = (rsem0, rsem1)
    isem = (isem0, isem1)

    def _zrow(j, _):
        for q in range(D // 16):
            rows0[j, pl.ds(q * 16, 16)] = jnp.zeros((16,), jnp.float32)
        return 0
    lax.fori_loop(0, ZROWS, _zrow, 0)
    for t in range((NZCHUNK + NS - 1) // NS):
        cid = t * NS + s
        @pl.when(cid < NZCHUNK)
        def _():
            pltpu.sync_copy(rows0, acc.at[pl.ds(cid * ZROWS, ZROWS)])
    plsc.subcore_barrier()

    def _issue_gr(i, sl):
        pltpu.async_copy(h_hbm.at[srcb[sl]], rows[sl], gsem[sl])
        pltpu.async_copy(rad_hbm.at[pl.ds(rbase + i * (K // 2), K // 2)], radb[sl], rsem[sl])

    def _issue_idx(i, sl):
        eb = tbase + i * K
        pltpu.async_copy(src_hbm.at[pl.ds(eb, K)], srcb[sl], isem[sl])
        pltpu.async_copy(dst_hbm.at[pl.ds(eb, K)], dstb[sl], isem[sl])

    pltpu.sync_copy(src_hbm.at[pl.ds(tbase, K)], srcb0)
    pltpu.sync_copy(dst_hbm.at[pl.ds(tbase, K)], dstb0)
    _issue_idx(1, 1)
    _issue_gr(0, 0)

    def _step(i, sl):
        nsl = 1 - sl
        pltpu.make_async_copy(h_hbm.at[srcb[sl]], rows[sl], gsem[sl]).wait()
        pltpu.make_async_copy(rad_hbm.at[pl.ds(rbase, K // 2)], radb[sl], rsem[sl]).wait()

        @pl.when(i + 1 < NCHUNK)
        def _():
            pltpu.make_async_copy(src_hbm.at[pl.ds(tbase, K)], srcb[nsl], isem[nsl]).wait()
            pltpu.make_async_copy(dst_hbm.at[pl.ds(tbase, K)], dstb[nsl], isem[nsl]).wait()
            _issue_gr(i + 1, nsl)

        @plsc.parallel_loop(0, K // 2, unroll=4)
        def _mul(jp):
            for q in range(D // 16):
                qs = pl.ds(q * 16, 16)
                rv = radb[sl][jp, qs]
                ra = jax.lax.bitcast_convert_type(jnp.left_shift(rv, 16), jnp.float32)
                rb = jax.lax.bitcast_convert_type(jnp.bitwise_and(rv, jnp.int32(-65536)), jnp.float32)
                rows[sl][2 * jp, qs] = rows[sl][2 * jp, qs] * ra
                rows[sl][2 * jp + 1, qs] = rows[sl][2 * jp + 1, qs] * rb
        pltpu.sync_copy(rows[sl], acc.at[dstb[sl]], add=True)

        @pl.when(i + 2 < NCHUNK)
        def _():
            _issue_idx(i + 2, sl)

    def _chunk(i, _):
        @pl.when(i % 2 == 0)
        def _():
            _step(i, 0)

        @pl.when(i % 2 == 1)
        def _():
            _step(i, 1)
        return 0
    lax.fori_loop(0, NCHUNK, _chunk, 0)

    plsc.subcore_barrier()
    for t in range((NFCHUNK + NS - 1) // NS):
        cid = t * NS + s
        @pl.when(cid < NFCHUNK)
        def _():
            r0 = cid * FROWS
            pltpu.sync_copy(acc.at[pl.ds(r0, FROWS)], out_hbm.at[pl.ds(c * N + r0, FROWS)])


def kernel(node_features, edge_index, edge_attrs, edge_embedding, W1, W2, W_sc, Wr1, Wr2):
    x = node_features
    src = edge_index[0]
    dst = edge_index[1]

    h = pl.pallas_call(
        _h_body,
        grid=(5,),
        in_specs=[pl.BlockSpec((2000, D), lambda i: (i, 0)),
                  pl.BlockSpec((D, D), lambda i: (0, 0))],
        out_specs=pl.BlockSpec((2000, D), lambda i: (i, 0)),
        out_shape=jax.ShapeDtypeStruct((N, D), jnp.float32),
    )(x, W1)

    BE = 8000
    radial = pl.pallas_call(
        _radial_body,
        grid=(E // BE,),
        in_specs=[pl.BlockSpec((BE, 16), lambda i: (i, 0)),
                  pl.BlockSpec((BE, 1), lambda i: (i, 0)),
                  pl.BlockSpec((16, 64), lambda i: (0, 0)),
                  pl.BlockSpec((64, D), lambda i: (0, 0))],
        out_specs=pl.BlockSpec((BE // 2, D), lambda i: (i, 0)),
        out_shape=jax.ShapeDtypeStruct((E // 2, D), jnp.int32),
    )(edge_embedding, edge_attrs, Wr1, Wr2)

    agg2 = pl.kernel(
        _edge_body,
        out_type=jax.ShapeDtypeStruct((2 * N, D), jnp.float32),
        mesh=plsc.VectorSubcoreMesh(core_axis_name="c", subcore_axis_name="s"),
        scratch_types=[
            pltpu.VMEM((K,), jnp.int32),
            pltpu.VMEM((K,), jnp.int32),
            pltpu.VMEM((K,), jnp.int32),
            pltpu.VMEM((K,), jnp.int32),
            pltpu.VMEM((K, D), jnp.float32),
            pltpu.VMEM((K, D), jnp.float32),
            pltpu.VMEM((K // 2, D), jnp.int32),
            pltpu.VMEM((K // 2, D), jnp.int32),
            pltpu.VMEM_SHARED((N, D), jnp.float32),
            pltpu.SemaphoreType.DMA,
            pltpu.SemaphoreType.DMA,
            pltpu.SemaphoreType.DMA,
            pltpu.SemaphoreType.DMA,
            pltpu.SemaphoreType.DMA,
            pltpu.SemaphoreType.DMA,
        ],
    )(h, src, dst, radial)

    out = pl.pallas_call(
        _post_body,
        grid=(5,),
        in_specs=[pl.BlockSpec((2000, D), lambda i: (i, 0)),
                  pl.BlockSpec((2000, D), lambda i: (i, 0)),
                  pl.BlockSpec((2000, D), lambda i: (i, 0)),
                  pl.BlockSpec((D, D), lambda i: (0, 0)),
                  pl.BlockSpec((D, D), lambda i: (0, 0))],
        out_specs=pl.BlockSpec((2000, D), lambda i: (i, 0)),
        out_shape=jax.ShapeDtypeStruct((N, D), jnp.float32),
    )(x, agg2[:N], agg2[N:], W2, W_sc)
    return out
